# baseline (speedup 1.0000x reference)
import functools

import jax
import jax.numpy as jnp
from jax import lax
from jax.experimental import pallas as pl
from jax.experimental.pallas import tpu as pltpu
from jax.experimental.pallas import tpu_sc as plsc

_B, _C, _D, _H, _W = 2, 1, 3, 512, 512
_BONUS = 10.0
_EPS = 1e-07
_NW = 32
_RPW = _H // _NW
_RB = 8
_NCH = _W // 16

_mesh = plsc.VectorSubcoreMesh(core_axis_name="c", subcore_axis_name="s")


@functools.partial(
    pl.kernel,
    mesh=_mesh,
    out_type=[
        jax.ShapeDtypeStruct((_B * _D, _H, _W), jnp.float32),
        jax.ShapeDtypeStruct((_B, 3, _D, _H, _W), jnp.float32),
    ],
    scratch_types=[
        pltpu.VMEM((3, _D, _RB + 8, _W), jnp.float32),
        pltpu.VMEM((_D, _RB, _W), jnp.float32),
        pltpu.VMEM((_D, 3, _RB, _W), jnp.float32),
    ],
)
def _sc_kernel(xs_hbm, y_hbm, c_hbm, pin, ybuf, cbuf):
    f32 = jnp.float32
    ninf = f32(-jnp.inf)
    wid = lax.axis_index("s") * 2 + lax.axis_index("c")
    iota = lax.iota(jnp.int32, 16)

    for b in range(_B):
        for half in range(_RPW // _RB):
            row0 = wid * _RPW + half * _RB
            for s in range(3):
                for q in range(_D):
                    pltpu.sync_copy(
                        xs_hbm.at[s, b * _D + q, pl.ds(row0, _RB + 8), :],
                        pin.at[s, q],
                    )

            def row_body(r, _):
                grv = jnp.full((16,), row0 + r, jnp.int32)
                tmask = jnp.where(row0 + r == 0, ninf, f32(0.0))
                bmask = jnp.where(row0 + r == (_H - 1), ninf, f32(0.0))

                @plsc.parallel_loop(0, _NCH, step=1, unroll=2)
                def chunk_body(cidx):
                    col = pl.multiple_of(cidx * 16, 16)
                    wv = col + iota
                    mlo = jnp.where(wv == 0, ninf, f32(0.0))
                    mhi = jnp.where(wv == (_W - 1), ninf, f32(0.0))

                    def ml(a):
                        return a + mlo

                    def mr(a):
                        return a + mhi

                    v = [
                        [
                            [pin[dw, q, r + dr, pl.ds(col, 16)] for dw in range(3)]
                            for dr in range(3)
                        ]
                        for q in range(_D)
                    ]
                    n8, n9 = [], []
                    for q in range(_D):
                        hu = jnp.maximum(
                            jnp.maximum(ml(v[q][0][0]), v[q][0][1]),
                            mr(v[q][0][2]),
                        )
                        hd = jnp.maximum(
                            jnp.maximum(ml(v[q][2][0]), v[q][2][1]),
                            mr(v[q][2][2]),
                        )
                        hum = hu + tmask
                        hdm = hd + bmask
                        hc = jnp.maximum(ml(v[q][1][0]), mr(v[q][1][2]))
                        e8 = jnp.maximum(jnp.maximum(hum, hdm), hc)
                        n8.append(e8)
                        n9.append(jnp.maximum(e8, v[q][1][1]))
                    for d in range(_D):
                        pm = max(d - 1, 0)
                        pp = min(d + 1, _D - 1)
                        xc = v[d][1][1]
                        gx = 0.5 * (v[d][1][2] - v[d][1][0])
                        gy = 0.5 * (v[d][2][1] - v[d][0][1])
                        gs = 0.5 * (v[pp][1][1] - v[pm][1][1])
                        dxx = v[d][1][2] + v[d][1][0] - 2.0 * xc
                        dyy = v[d][2][1] + v[d][0][1] - 2.0 * xc
                        dss = v[pp][1][1] + v[pm][1][1] - 2.0 * xc
                        dxy = 0.25 * (
                            (v[d][2][2] - v[d][2][0]) - (v[d][0][2] - v[d][0][0])
                        )
                        dys = 0.25 * (
                            (v[pp][2][1] - v[pp][0][1]) - (v[pm][2][1] - v[pm][0][1])
                        )
                        dxs = 0.25 * (
                            (v[pp][1][2] - v[pp][1][0]) - (v[pm][1][2] - v[pm][1][0])
                        )
                        neigh = n8[d]
                        if d > 0:
                            neigh = jnp.maximum(neigh, n9[d - 1])
                        if d < _D - 1:
                            neigh = jnp.maximum(neigh, n9[d + 1])
                        nmsf = jnp.where(xc > neigh, f32(1.0), f32(0.0))
                        a_ = dyy * dss - dys * dys
                        b_ = dxs * dys - dxy * dss
                        c_ = dxy * dys - dxs * dyy
                        d_ = dxx * dss - dxs * dxs
                        e_ = dxy * dxs - dxx * dys
                        f_ = dxx * dyy - dxy * dxy
                        det = dxx * a_ + dxy * b_ + dxs * c_
                        validf = jnp.where(jnp.abs(det) > _EPS, f32(1.0), f32(0.0))
                        rdet = 1.0 / det
                        s0 = (a_ * gx + b_ * gy + c_ * gs) * rdet
                        s1 = (b_ * gx + d_ * gy + e_ * gs) * rdet
                        s2 = (c_ * gx + e_ * gy + f_ * gs) * rdet
                        nvf = nmsf * validf
                        amax = jnp.maximum(
                            jnp.maximum(jnp.abs(s0), jnp.abs(s1)), jnp.abs(s2)
                        )
                        mf = nvf * jnp.where(amax <= 0.7, f32(1.0), f32(0.0))
                        m = mf > f32(0.5)
                        d0 = jnp.where(m, -s0, f32(0.0))
                        d1 = jnp.where(m, -s1, f32(0.0))
                        d2 = jnp.where(m, -s2, f32(0.0))
                        t = gx * s0 + gy * s1 + gs * s2
                        dy_ = jnp.where(m, -0.5 * t, f32(0.0))
                        yv = (
                            xc
                            + dy_
                            + jnp.where(nvf > f32(0.5), f32(_BONUS), f32(0.0))
                        )
                        ybuf[d, r, pl.ds(col, 16)] = yv
                        cbuf[d, 0, r, pl.ds(col, 16)] = f32(d) + d2
                        cbuf[d, 1, r, pl.ds(col, 16)] = grv.astype(f32) + d1
                        cbuf[d, 2, r, pl.ds(col, 16)] = wv.astype(f32) + d0

                return 0

            lax.fori_loop(0, _RB, row_body, 0)
            for d in range(_D):
                pltpu.sync_copy(
                    ybuf.at[d], y_hbm.at[b * _D + d, pl.ds(row0, _RB), :]
                )
                for comp in range(3):
                    pltpu.sync_copy(
                        cbuf.at[d, comp],
                        c_hbm.at[b, comp, d, pl.ds(row0, _RB), :],
                    )


def kernel(x):
    xr = x.reshape(_B * _D, _H, _W)
    xpe = jnp.pad(xr, ((0, 0), (1, 7), (1, 1)), mode="edge")
    xs = jnp.stack(
        [xpe[:, :, 0:_W], xpe[:, :, 1 : _W + 1], xpe[:, :, 2 : _W + 2]], axis=0
    )
    y, coords = _sc_kernel(xs)
    return (
        coords.reshape(_B, _C, 3, _D, _H, _W),
        y.reshape(_B, _C, _D, _H, _W),
    )


# speedup vs baseline: 1.0275x; 1.0275x over previous
import functools

import jax
import jax.numpy as jnp
from jax import lax
from jax.experimental import pallas as pl
from jax.experimental.pallas import tpu as pltpu
from jax.experimental.pallas import tpu_sc as plsc

_B, _C, _D, _H, _W = 2, 1, 3, 512, 512
_BONUS = 10.0
_EPS = 1e-07
_NW = 32
_RPW = _H // _NW
_RB = 8
_NCH = _W // 16

_mesh = plsc.VectorSubcoreMesh(core_axis_name="c", subcore_axis_name="s")


@functools.partial(
    pl.kernel,
    mesh=_mesh,
    out_type=[
        jax.ShapeDtypeStruct((_B * _D, _H, _W), jnp.float32),
        jax.ShapeDtypeStruct((_B, 3, _D, _H, _W), jnp.float32),
    ],
    scratch_types=[
        pltpu.VMEM((3, _D, _RB + 8, _W), jnp.float32),
        pltpu.VMEM((_D, _RB, _W), jnp.float32),
        pltpu.VMEM((_D, 3, _RB, _W), jnp.float32),
    ],
)
def _sc_kernel(xs_hbm, y_hbm, c_hbm, pin, ybuf, cbuf):
    f32 = jnp.float32
    ninf = f32(-jnp.inf)
    wid = lax.axis_index("s") * 2 + lax.axis_index("c")
    iota = lax.iota(jnp.int32, 16)

    for b in range(_B):
        for half in range(_RPW // _RB):
            row0 = wid * _RPW + half * _RB
            for s in range(3):
                for q in range(_D):
                    pltpu.sync_copy(
                        xs_hbm.at[s, b * _D + q, pl.ds(row0, _RB + 8), :],
                        pin.at[s, q],
                    )

            def row_body(r, _):
                grvf = jnp.full((16,), row0 + r, jnp.int32).astype(f32)
                tmask = jnp.where(row0 + r == 0, ninf, f32(0.0))
                bmask = jnp.where(row0 + r == (_H - 1), ninf, f32(0.0))

                def chunk_body(cidx, _):
                    col = pl.multiple_of(cidx * 16, 16)
                    wv = col + iota
                    mlo = jnp.where(wv == 0, ninf, f32(0.0))
                    mhi = jnp.where(wv == (_W - 1), ninf, f32(0.0))

                    v = [
                        [
                            [pin[dw, q, r + dr, pl.ds(col, 16)] for dw in range(3)]
                            for dr in range(3)
                        ]
                        for q in range(_D)
                    ]
                    n8, n9, gxq, gyq, cs = [], [], [], [], []
                    for q in range(_D):
                        hu = jnp.maximum(
                            jnp.maximum(v[q][0][0], v[q][0][1]), v[q][0][2]
                        )
                        hd = jnp.maximum(
                            jnp.maximum(v[q][2][0], v[q][2][1]), v[q][2][2]
                        )
                        hum = hu + tmask
                        hdm = hd + bmask
                        hc = jnp.maximum(v[q][1][0] + mlo, v[q][1][2] + mhi)
                        e8 = jnp.maximum(jnp.maximum(hum, hdm), hc)
                        n8.append(e8)
                        n9.append(jnp.maximum(e8, v[q][1][1]))
                        gxq.append(0.5 * (v[q][1][2] - v[q][1][0]))
                        gyq.append(0.5 * (v[q][2][1] - v[q][0][1]))
                        cs.append(v[q][1][1])
                    d01 = cs[1] - cs[0]
                    d21 = cs[2] - cs[1]
                    gs3 = (0.5 * d01, 0.5 * (d21 + d01), 0.5 * d21)
                    dss3 = (d01, d21 - d01, -d21)
                    for d in range(_D):
                        pm = max(d - 1, 0)
                        pp = min(d + 1, _D - 1)
                        xc = cs[d]
                        gx = gxq[d]
                        gy = gyq[d]
                        gs = gs3[d]
                        xc2 = 2.0 * xc
                        dxx = v[d][1][2] + v[d][1][0] - xc2
                        dyy = v[d][2][1] + v[d][0][1] - xc2
                        dss = dss3[d]
                        dxy = 0.25 * (
                            (v[d][2][2] - v[d][2][0]) - (v[d][0][2] - v[d][0][0])
                        )
                        dys = 0.5 * (gyq[pp] - gyq[pm])
                        dxs = 0.5 * (gxq[pp] - gxq[pm])
                        neigh = n8[d]
                        if d > 0:
                            neigh = jnp.maximum(neigh, n9[d - 1])
                        if d < _D - 1:
                            neigh = jnp.maximum(neigh, n9[d + 1])
                        nmsf = jnp.where(xc > neigh, f32(1.0), f32(0.0))
                        a_ = dyy * dss - dys * dys
                        b_ = dxs * dys - dxy * dss
                        c_ = dxy * dys - dxs * dyy
                        d_ = dxx * dss - dxs * dxs
                        e_ = dxy * dxs - dxx * dys
                        f_ = dxx * dyy - dxy * dxy
                        det = dxx * a_ + dxy * b_ + dxs * c_
                        validf = jnp.where(jnp.abs(det) > _EPS, f32(1.0), f32(0.0))
                        rdet = 1.0 / det
                        s0 = (a_ * gx + b_ * gy + c_ * gs) * rdet
                        s1 = (b_ * gx + d_ * gy + e_ * gs) * rdet
                        s2 = (c_ * gx + e_ * gy + f_ * gs) * rdet
                        nvf = nmsf * validf
                        amax = jnp.maximum(
                            jnp.maximum(jnp.abs(s0), jnp.abs(s1)), jnp.abs(s2)
                        )
                        mf = nvf * jnp.where(amax <= 0.7, f32(1.0), f32(0.0))
                        m = mf > f32(0.5)
                        d0 = jnp.where(m, -s0, f32(0.0))
                        d1 = jnp.where(m, -s1, f32(0.0))
                        d2 = jnp.where(m, -s2, f32(0.0))
                        t = gx * s0 + gy * s1 + gs * s2
                        dy_ = jnp.where(m, -0.5 * t, f32(0.0))
                        yv = (
                            xc
                            + dy_
                            + jnp.where(nvf > f32(0.5), f32(_BONUS), f32(0.0))
                        )
                        ybuf[d, r, pl.ds(col, 16)] = yv
                        cbuf[d, 0, r, pl.ds(col, 16)] = f32(d) + d2
                        cbuf[d, 1, r, pl.ds(col, 16)] = grvf + d1
                        cbuf[d, 2, r, pl.ds(col, 16)] = wv.astype(f32) + d0
                    return 0

                lax.fori_loop(0, _NCH, chunk_body, 0)
                return 0

            lax.fori_loop(0, _RB, row_body, 0)
            for d in range(_D):
                pltpu.sync_copy(
                    ybuf.at[d], y_hbm.at[b * _D + d, pl.ds(row0, _RB), :]
                )
                for comp in range(3):
                    pltpu.sync_copy(
                        cbuf.at[d, comp],
                        c_hbm.at[b, comp, d, pl.ds(row0, _RB), :],
                    )


def kernel(x):
    xr = x.reshape(_B * _D, _H, _W)
    xpe = jnp.pad(xr, ((0, 0), (1, 7), (1, 1)), mode="edge")
    xs = jnp.stack(
        [xpe[:, :, 0:_W], xpe[:, :, 1 : _W + 1], xpe[:, :, 2 : _W + 2]], axis=0
    )
    y, coords = _sc_kernel(xs)
    return (
        coords.reshape(_B, _C, 3, _D, _H, _W),
        y.reshape(_B, _C, _D, _H, _W),
    )
